# baseline (speedup 1.0000x reference)
import functools

import jax
import jax.numpy as jnp
from jax.experimental import pallas as pl
from jax.experimental.pallas import tpu as pltpu

_NUM_EXPERTS = 64
_TOP_K = 8
_ROWS = 512


def _route_block(x, w, gate_ref, idx_ref, psum_ref, fsum_ref):
    logits = jax.lax.dot_general(
        w, x, (((1,), (1,)), ((), ())), preferred_element_type=jnp.float32)

    m = jnp.max(logits, axis=0, keepdims=True)
    e = jnp.exp(logits - m)
    s = jnp.sum(e, axis=0, keepdims=True)
    probs = e / s

    iota = jax.lax.broadcasted_iota(jnp.int32, probs.shape, 0)
    p = probs
    vals = []
    idxs = []
    for _ in range(_TOP_K):
        mv = jnp.max(p, axis=0, keepdims=True)
        ij = jnp.min(jnp.where(p == mv, iota, _NUM_EXPERTS), axis=0,
                     keepdims=True)
        vals.append(mv)
        idxs.append(ij)
        p = jnp.where(iota == ij, -1.0, p)
    v = jnp.concatenate(vals, axis=0)
    gate_ref[...] = v / jnp.sum(v, axis=0, keepdims=True)
    idx_ref[...] = jnp.concatenate(idxs, axis=0)

    mask = (p < 0).astype(jnp.float32)
    psum_ref[...] += jnp.sum(probs, axis=1, keepdims=True)
    fsum_ref[...] += jnp.sum(mask, axis=1, keepdims=True)


def _router_body(xa_ref, xb_ref, w_ref, gate_a_ref, idx_a_ref, gate_b_ref,
                 idx_b_ref, aux_ref, psum_ref, fsum_ref, *, n_tokens):
    step = pl.program_id(0)
    nsteps = pl.num_programs(0)

    @pl.when(step == 0)
    def _init():
        psum_ref[...] = jnp.zeros_like(psum_ref)
        fsum_ref[...] = jnp.zeros_like(fsum_ref)

    w = w_ref[...]
    _route_block(xa_ref[...], w, gate_a_ref, idx_a_ref, psum_ref, fsum_ref)
    _route_block(xb_ref[...], w, gate_b_ref, idx_b_ref, psum_ref, fsum_ref)

    @pl.when(step == nsteps - 1)
    def _finalize():
        f = fsum_ref[...] / n_tokens
        pbar = psum_ref[...] / n_tokens
        aux_ref[...] = jnp.sum(_NUM_EXPERTS * f * pbar, keepdims=True
                               ).reshape(1, 1)


def kernel(x, W):
    b, s, d = x.shape
    n = b * s
    half = n // 2
    xf = x.reshape(n, d)
    grid = half // _ROWS
    nblk = grid
    gate_a, idx_a, gate_b, idx_b, aux = pl.pallas_call(
        functools.partial(_router_body, n_tokens=n),
        grid=(grid,),
        in_specs=[
            pl.BlockSpec((_ROWS, d), lambda i: (i, 0)),
            pl.BlockSpec((_ROWS, d), lambda i, _n=nblk: (i + _n, 0)),
            pl.BlockSpec((_NUM_EXPERTS, d), lambda i: (0, 0)),
        ],
        out_specs=[
            pl.BlockSpec((_TOP_K, _ROWS), lambda i: (0, i)),
            pl.BlockSpec((_TOP_K, _ROWS), lambda i: (0, i)),
            pl.BlockSpec((_TOP_K, _ROWS), lambda i: (0, i)),
            pl.BlockSpec((_TOP_K, _ROWS), lambda i: (0, i)),
            pl.BlockSpec((1, 1), lambda i: (0, 0)),
        ],
        out_shape=[
            jax.ShapeDtypeStruct((_TOP_K, half), jnp.float32),
            jax.ShapeDtypeStruct((_TOP_K, half), jnp.int32),
            jax.ShapeDtypeStruct((_TOP_K, half), jnp.float32),
            jax.ShapeDtypeStruct((_TOP_K, half), jnp.int32),
            jax.ShapeDtypeStruct((1, 1), jnp.float32),
        ],
        scratch_shapes=[
            pltpu.VMEM((_NUM_EXPERTS, 1), jnp.float32),
            pltpu.VMEM((_NUM_EXPERTS, 1), jnp.float32),
        ],
        compiler_params=pltpu.CompilerParams(
            dimension_semantics=("arbitrary",)),
    )(xf, xf, W)
    gate_t = jnp.concatenate([gate_a, gate_b], axis=1)
    idx_t = jnp.concatenate([idx_a, idx_b], axis=1)
    return gate_t.T.astype(x.dtype), idx_t.T, aux.reshape(())


# speedup vs baseline: 1.0719x; 1.0719x over previous
import functools

import jax
import jax.numpy as jnp
from jax.experimental import pallas as pl
from jax.experimental.pallas import tpu as pltpu

_NUM_EXPERTS = 64
_TOP_K = 8
_ROWS = 1024


def _router_body(x_ref, w_ref, gate_ref, idx_ref, aux_ref, psum_ref, fsum_ref,
                 *, n_tokens):
    step = pl.program_id(0)
    nsteps = pl.num_programs(0)

    @pl.when(step == 0)
    def _init():
        psum_ref[...] = jnp.zeros_like(psum_ref)
        fsum_ref[...] = jnp.zeros_like(fsum_ref)

    x = x_ref[...]
    w = w_ref[...]
    logits = jax.lax.dot_general(
        w, x, (((1,), (1,)), ((), ())), preferred_element_type=jnp.float32)

    m = jnp.max(logits, axis=0, keepdims=True)
    e = jnp.exp(logits - m)
    s = jnp.sum(e, axis=0, keepdims=True)
    probs = e / s

    iota = jax.lax.broadcasted_iota(jnp.int32, probs.shape, 0)
    p = probs
    vals = []
    idxs = []
    for _ in range(_TOP_K):
        mv = jnp.max(p, axis=0, keepdims=True)
        ij = jnp.min(jnp.where(p == mv, iota, _NUM_EXPERTS), axis=0,
                     keepdims=True)
        vals.append(mv)
        idxs.append(ij)
        p = jnp.where(iota == ij, -1.0, p)
    v = jnp.concatenate(vals, axis=0)
    gate_ref[...] = v / jnp.sum(v, axis=0, keepdims=True)
    idx_ref[...] = jnp.concatenate(idxs, axis=0)

    mask = (p < 0).astype(jnp.float32)
    psum_ref[...] += jnp.sum(probs, axis=1, keepdims=True)
    fsum_ref[...] += jnp.sum(mask, axis=1, keepdims=True)

    @pl.when(step == nsteps - 1)
    def _finalize():
        f = fsum_ref[...] / n_tokens
        pbar = psum_ref[...] / n_tokens
        aux_ref[...] = jnp.sum(_NUM_EXPERTS * f * pbar, keepdims=True
                               ).reshape(1, 1)


def kernel(x, W):
    b, s, d = x.shape
    n = b * s
    xf = x.reshape(n, d)
    grid = n // _ROWS
    gate_t, idx_t, aux = pl.pallas_call(
        functools.partial(_router_body, n_tokens=n),
        grid=(grid,),
        in_specs=[
            pl.BlockSpec((_ROWS, d), lambda i: (i, 0)),
            pl.BlockSpec((_NUM_EXPERTS, d), lambda i: (0, 0)),
        ],
        out_specs=[
            pl.BlockSpec((_TOP_K, _ROWS), lambda i: (0, i)),
            pl.BlockSpec((_TOP_K, _ROWS), lambda i: (0, i)),
            pl.BlockSpec((1, 1), lambda i: (0, 0)),
        ],
        out_shape=[
            jax.ShapeDtypeStruct((_TOP_K, n), jnp.float32),
            jax.ShapeDtypeStruct((_TOP_K, n), jnp.int32),
            jax.ShapeDtypeStruct((1, 1), jnp.float32),
        ],
        scratch_shapes=[
            pltpu.VMEM((_NUM_EXPERTS, 1), jnp.float32),
            pltpu.VMEM((_NUM_EXPERTS, 1), jnp.float32),
        ],
        compiler_params=pltpu.CompilerParams(
            dimension_semantics=("arbitrary",)),
    )(xf, W)
    return gate_t.T.astype(x.dtype), idx_t.T, aux.reshape(())
